# baseline (speedup 1.0000x reference)
import functools

import jax
import jax.numpy as jnp
from jax import lax
from jax.experimental import pallas as pl
from jax.experimental.pallas import tpu as pltpu
from jax.experimental.pallas import tpu_sc as plsc

N = 10000
E = 320000
D = 128
R = 5

NC = 2
NS = 16
NW = NC * NS
L = 16

EPT = E // NW
B = 128
NB_FULL = EPT // B
TAIL = EPT - NB_FULL * B
E_PAD = NW * EPT + B

RN = R * N
RNP = 50176
DEG_TOTAL = 2 * RNP
TRASH = RN
ZROW = N

_mesh = plsc.VectorSubcoreMesh(core_axis_name="c", subcore_axis_name="s")


DEG_CHUNK = DEG_TOTAL // NS
DEG_NCOPY = DEG_CHUNK // B


@functools.partial(
    pl.kernel,
    out_type=jax.ShapeDtypeStruct((NC, DEG_TOTAL), jnp.float32),
    mesh=_mesh,
    scratch_types=[
        pltpu.VMEM((3, B), jnp.int32),
        pltpu.VMEM((2, B), jnp.int32),
        pltpu.VMEM((2, B), jnp.float32),
        pltpu.VMEM((B,), jnp.float32),
        pltpu.VMEM_SHARED((DEG_TOTAL,), jnp.float32),
    ],
)
def _deg(src_hbm, dst_hbm, rate_hbm, out_hbm, ev, keys, ones, zbuf, acc):
    c = lax.axis_index("c")
    s = lax.axis_index("s")
    wid = s * NC + c

    z16 = jnp.zeros((L,), jnp.float32)
    o16 = jnp.ones((L,), jnp.float32)
    for j in range(B // L):
        sl = pl.ds(j * L, L)
        ones[0, sl] = o16
        ones[1, sl] = o16
        zbuf[sl] = z16

    def zero_body(t, carry):
        pltpu.sync_copy(zbuf, acc.at[pl.ds(s * DEG_CHUNK + t * B, B)])
        return carry

    lax.fori_loop(0, DEG_NCOPY, zero_body, 0)
    plsc.subcore_barrier()

    base = wid * EPT

    def do_batch(i, tail):
        off = base + i * B
        pltpu.sync_copy(src_hbm.at[pl.ds(off, B)], ev.at[0])
        pltpu.sync_copy(dst_hbm.at[pl.ds(off, B)], ev.at[1])
        pltpu.sync_copy(rate_hbm.at[pl.ds(off, B)], ev.at[2])
        for j in range(B // L):
            sl = pl.ds(j * L, L)
            if tail and j * L >= TAIL:
                t16 = jnp.full((L,), TRASH, jnp.int32)
                keys[0, sl] = t16
                keys[1, sl] = t16
            else:
                sv = ev[0, sl]
                dv = ev[1, sl]
                rv = ev[2, sl]
                rn = rv * N
                keys[0, sl] = rn + sv
                keys[1, sl] = rn + dv + RNP
        pltpu.sync_copy(ones.at[0], acc.at[keys.at[0]], add=True)
        pltpu.sync_copy(ones.at[1], acc.at[keys.at[1]], add=True)

    def batch_body(i, carry):
        do_batch(i, False)
        return carry

    lax.fori_loop(0, NB_FULL, batch_body, 0)
    do_batch(NB_FULL, True)

    plsc.subcore_barrier()

    def flush_body(t, carry):
        off = s * DEG_CHUNK + t * B
        pltpu.sync_copy(acc.at[pl.ds(off, B)], zbuf)
        pltpu.sync_copy(zbuf, out_hbm.at[c, pl.ds(off, B)])
        return carry

    lax.fori_loop(0, DEG_NCOPY, flush_body, 0)


NP = 10240
ROWS_PT = NP // NS
ZB_ROWS = 80


def _make_agg(r):
    @functools.partial(
        pl.kernel,
        out_type=jax.ShapeDtypeStruct((NC, NP, D), jnp.float32),
        mesh=_mesh,
        scratch_types=[
            pltpu.VMEM((3, B), jnp.int32),
            pltpu.VMEM((B,), jnp.int32),
            pltpu.VMEM((B, D), jnp.float32),
            pltpu.VMEM((ZB_ROWS, D), jnp.float32),
            pltpu.VMEM_SHARED((NP, D), jnp.float32),
        ],
    )
    def _agg(hsrc_hbm, src_hbm, dst_hbm, rate_hbm, out_hbm,
             ev, gidx, rows, fbuf, acc):
        c = lax.axis_index("c")
        s = lax.axis_index("s")
        wid = s * NC + c

        z16 = jnp.zeros((L,), jnp.float32)
        for q in range(ZB_ROWS):
            for j in range(D // L):
                fbuf[q, pl.ds(j * L, L)] = z16

        rbase = s * ROWS_PT
        for t in range(ROWS_PT // ZB_ROWS):
            pltpu.sync_copy(fbuf, acc.at[pl.ds(rbase + t * ZB_ROWS, ZB_ROWS)])
        plsc.subcore_barrier()

        base = wid * EPT

        def do_batch(i, tail):
            off = base + i * B
            pltpu.sync_copy(src_hbm.at[pl.ds(off, B)], ev.at[0])
            pltpu.sync_copy(dst_hbm.at[pl.ds(off, B)], ev.at[1])
            pltpu.sync_copy(rate_hbm.at[pl.ds(off, B)], ev.at[2])
            for j in range(B // L):
                sl = pl.ds(j * L, L)
                if tail and j * L >= TAIL:
                    gidx[sl] = jnp.full((L,), ZROW, jnp.int32)
                else:
                    sv = ev[0, sl]
                    rv = ev[2, sl]
                    gidx[sl] = jnp.where(rv == r, sv, ZROW)
            pltpu.sync_copy(hsrc_hbm.at[gidx], rows)
            pltpu.sync_copy(rows, acc.at[ev.at[1]], add=True)

        def batch_body(i, carry):
            do_batch(i, False)
            return carry

        lax.fori_loop(0, NB_FULL, batch_body, 0)
        do_batch(NB_FULL, True)

        plsc.subcore_barrier()

        for t in range(ROWS_PT // ZB_ROWS):
            off = rbase + t * ZB_ROWS
            pltpu.sync_copy(acc.at[pl.ds(off, ZB_ROWS)], fbuf)
            pltpu.sync_copy(fbuf, out_hbm.at[c, pl.ds(off, ZB_ROWS)])

    return _agg


_agg_calls = [_make_agg(r) for r in range(R)]


def _norms_body(d_ref, o_ref):
    o_ref[...] = lax.rsqrt(jnp.maximum(d_ref[0] + d_ref[1], 1.0))


_norms_call = pl.pallas_call(
    _norms_body,
    out_shape=jax.ShapeDtypeStruct((DEG_TOTAL // D, D), jnp.float32),
)

BR = 1000


def _scale_body(x_ref, n_ref, o_ref):
    o_ref[...] = x_ref[...] * n_ref[...]


_scale_call = pl.pallas_call(
    _scale_body,
    grid=(N // BR,),
    in_specs=[
        pl.BlockSpec((BR, D), lambda i: (i, 0)),
        pl.BlockSpec((BR, 1), lambda i: (i, 0)),
    ],
    out_specs=pl.BlockSpec((BR, D), lambda i: (i, 0)),
    out_shape=jax.ShapeDtypeStruct((N, D), jnp.float32),
)


def _mm_body(p_ref, nd_ref, w_ref, b_ref, nn_ref, rst_ref, hn_ref):
    a = (p_ref[0] + p_ref[1]) * nd_ref[...]
    v = jnp.dot(a, w_ref[...], preferred_element_type=jnp.float32) + b_ref[...]
    rst_ref[...] = v
    hn_ref[...] = v * nn_ref[...]


_mm_call = pl.pallas_call(
    _mm_body,
    grid=(N // BR,),
    in_specs=[
        pl.BlockSpec((2, BR, D), lambda i: (0, i, 0)),
        pl.BlockSpec((BR, 1), lambda i: (i, 0)),
        pl.BlockSpec((D, D), lambda i: (0, 0)),
        pl.BlockSpec((1, D), lambda i: (0, 0)),
        pl.BlockSpec((BR, 1), lambda i: (i, 0)),
    ],
    out_specs=[
        pl.BlockSpec((BR, D), lambda i: (i, 0)),
        pl.BlockSpec((BR, D), lambda i: (i, 0)),
    ],
    out_shape=[
        jax.ShapeDtypeStruct((N, D), jnp.float32),
        jax.ShapeDtypeStruct((N, D), jnp.float32),
    ],
)


def kernel(x, edge_index, edge_rate, W, b):
    src = edge_index[0]
    dst = edge_index[1]
    pad = E_PAD - E
    srcp = jnp.pad(src, (0, pad))
    dstp = jnp.pad(dst, (0, pad))
    ratep = jnp.pad(edge_rate, (0, pad), constant_values=R)

    deg = _deg(srcp, dstp, ratep)
    norm = _norms_call(deg.reshape(NC, DEG_TOTAL // D, D)).reshape(-1)
    nsrc = norm[0:RN].reshape(R, N)
    ndst = norm[RNP:RNP + RN].reshape(R, N)

    ones_col = jnp.ones((N, 1), jnp.float32)
    hsrc = _scale_call(x, nsrc[0].reshape(N, 1))
    outs = []
    for r in range(R):
        hsrc_pad = jnp.pad(hsrc, ((0, 1), (0, 0)))
        part = _agg_calls[r](hsrc_pad, srcp, dstp, ratep)
        nnext = nsrc[r + 1].reshape(N, 1) if r + 1 < R else ones_col
        rst, hsrc = _mm_call(part, ndst[r].reshape(N, 1), W[r],
                             b[r].reshape(1, D), nnext)
        outs.append(rst)
    return jnp.concatenate(outs, axis=1)
